# baseline (speedup 1.0000x reference)
import functools

import jax
import jax.numpy as jnp
from jax.experimental import pallas as pl
from jax.experimental.pallas import tpu as pltpu


def _adj_kernel(z1_ref, z2_ref, rk_ref, adj_ref, rk2_ref):
    a = z1_ref[0]
    b = z2_ref[0]
    logits = jax.lax.dot_general(
        a, b, (((1,), (1,)), ((), ())), preferred_element_type=jnp.float32
    )
    adj_ref[0] = jax.nn.sigmoid(logits)
    rk2_ref[...] = jax.nn.sigmoid(rk_ref[...])


@jax.jit
def kernel(z1, z2, rk_lgt):
    B, N, Z = z1.shape
    BM = 512
    BN = 2048
    grid = (B, N // BM, N // BN)

    adj, rk2 = pl.pallas_call(
        _adj_kernel,
        grid=grid,
        in_specs=[
            pl.BlockSpec((1, BM, Z), lambda b, i, j: (b, i, 0)),
            pl.BlockSpec((1, BN, Z), lambda b, i, j: (b, j, 0)),
            pl.BlockSpec((1, Z), lambda b, i, j: (0, 0)),
        ],
        out_specs=[
            pl.BlockSpec((1, BM, BN), lambda b, i, j: (b, i, j)),
            pl.BlockSpec((1, Z), lambda b, i, j: (0, 0)),
        ],
        out_shape=[
            jax.ShapeDtypeStruct((B, N, N), jnp.float32),
            jax.ShapeDtypeStruct(rk_lgt.shape, jnp.float32),
        ],
        compiler_params=pltpu.CompilerParams(
            dimension_semantics=("parallel", "parallel", "parallel"),
        ),
    )(z1, z2, rk_lgt)

    return (adj, z1, z2, rk2)


# speedup vs baseline: 1.1067x; 1.1067x over previous
import functools

import jax
import jax.numpy as jnp
from jax.experimental import pallas as pl
from jax.experimental.pallas import tpu as pltpu


def _adj_kernel(z1_ref, z2_ref, rk_ref, adj_ref, rk2_ref):
    a = z1_ref[0]
    b = z2_ref[0]
    logits = jax.lax.dot_general(
        a, b, (((1,), (1,)), ((), ())), preferred_element_type=jnp.float32
    )
    adj_ref[0] = 0.5 * jnp.tanh(0.5 * logits) + 0.5
    rk2_ref[...] = jax.nn.sigmoid(rk_ref[...])


@jax.jit
def kernel(z1, z2, rk_lgt):
    B, N, Z = z1.shape
    BM = 512
    BN = 2048
    grid = (B, N // BM, N // BN)

    adj, rk2 = pl.pallas_call(
        _adj_kernel,
        grid=grid,
        in_specs=[
            pl.BlockSpec((1, BM, Z), lambda b, i, j: (b, i, 0)),
            pl.BlockSpec((1, BN, Z), lambda b, i, j: (b, j, 0)),
            pl.BlockSpec((1, Z), lambda b, i, j: (0, 0)),
        ],
        out_specs=[
            pl.BlockSpec((1, BM, BN), lambda b, i, j: (b, i, j)),
            pl.BlockSpec((1, Z), lambda b, i, j: (0, 0)),
        ],
        out_shape=[
            jax.ShapeDtypeStruct((B, N, N), jnp.float32),
            jax.ShapeDtypeStruct(rk_lgt.shape, jnp.float32),
        ],
        compiler_params=pltpu.CompilerParams(
            dimension_semantics=("parallel", "parallel", "parallel"),
        ),
    )(z1, z2, rk_lgt)

    return (adj, z1, z2, rk2)


# speedup vs baseline: 1.2621x; 1.1405x over previous
import functools

import jax
import jax.numpy as jnp
from jax.experimental import pallas as pl
from jax.experimental.pallas import tpu as pltpu


def _adj_kernel(z1_ref, z2_ref, rk_ref, adj_ref, rk2_ref):
    a = z1_ref[0]
    b = z2_ref[0]
    logits = jax.lax.dot_general(
        a, b, (((1,), (1,)), ((), ())), preferred_element_type=jnp.float32
    )
    adj_ref[0] = 0.5 * jnp.tanh(0.5 * logits) + 0.5
    rk2_ref[...] = jax.nn.sigmoid(rk_ref[...])


@jax.jit
def kernel(z1, z2, rk_lgt):
    B, N, Z = z1.shape
    BM = 256
    BN = 4096
    grid = (B, N // BM, N // BN)

    adj, rk2 = pl.pallas_call(
        _adj_kernel,
        grid=grid,
        in_specs=[
            pl.BlockSpec((1, BM, Z), lambda b, i, j: (b, i, 0)),
            pl.BlockSpec((1, BN, Z), lambda b, i, j: (b, j, 0)),
            pl.BlockSpec((1, Z), lambda b, i, j: (0, 0)),
        ],
        out_specs=[
            pl.BlockSpec((1, BM, BN), lambda b, i, j: (b, i, j)),
            pl.BlockSpec((1, Z), lambda b, i, j: (0, 0)),
        ],
        out_shape=[
            jax.ShapeDtypeStruct((B, N, N), jnp.float32),
            jax.ShapeDtypeStruct(rk_lgt.shape, jnp.float32),
        ],
        compiler_params=pltpu.CompilerParams(
            dimension_semantics=("parallel", "parallel", "parallel"),
        ),
    )(z1, z2, rk_lgt)

    return (adj, z1, z2, rk2)


# speedup vs baseline: 1.4081x; 1.1157x over previous
import functools

import jax
import jax.numpy as jnp
from jax.experimental import pallas as pl
from jax.experimental.pallas import tpu as pltpu


def _adj_kernel(z1_ref, z2_ref, rk_ref, adj_ref, rk2_ref):
    a = z1_ref[0]
    b = z2_ref[0]
    logits = jax.lax.dot_general(
        a, b, (((1,), (1,)), ((), ())), preferred_element_type=jnp.float32
    )
    adj_ref[0] = 0.5 * jnp.tanh(0.5 * logits) + 0.5
    rk2_ref[...] = jax.nn.sigmoid(rk_ref[...])


@jax.jit
def kernel(z1, z2, rk_lgt):
    B, N, Z = z1.shape
    BM = 512
    BN = 4096
    grid = (B, N // BM, N // BN)

    adj, rk2 = pl.pallas_call(
        _adj_kernel,
        grid=grid,
        in_specs=[
            pl.BlockSpec((1, BM, Z), lambda b, i, j: (b, i, 0)),
            pl.BlockSpec((1, BN, Z), lambda b, i, j: (b, j, 0)),
            pl.BlockSpec((1, Z), lambda b, i, j: (0, 0)),
        ],
        out_specs=[
            pl.BlockSpec((1, BM, BN), lambda b, i, j: (b, i, j)),
            pl.BlockSpec((1, Z), lambda b, i, j: (0, 0)),
        ],
        out_shape=[
            jax.ShapeDtypeStruct((B, N, N), jnp.float32),
            jax.ShapeDtypeStruct(rk_lgt.shape, jnp.float32),
        ],
        compiler_params=pltpu.CompilerParams(
            dimension_semantics=("parallel", "parallel", "parallel"),
        ),
    )(z1, z2, rk_lgt)

    return (adj, z1, z2, rk2)


# speedup vs baseline: 1.4102x; 1.0015x over previous
import functools

import jax
import jax.numpy as jnp
from jax.experimental import pallas as pl
from jax.experimental.pallas import tpu as pltpu


def _adj_kernel(z1_ref, z2_ref, rk_ref, adj_ref, rk2_ref):
    a = z1_ref[0]
    b = z2_ref[0]
    logits = jax.lax.dot_general(
        a, b, (((1,), (1,)), ((), ())), preferred_element_type=jnp.float32
    )
    adj_ref[0] = 0.5 * jnp.tanh(0.5 * logits) + 0.5
    rk2_ref[...] = jax.nn.sigmoid(rk_ref[...])


@jax.jit
def kernel(z1, z2, rk_lgt):
    B, N, Z = z1.shape
    BM = 1024
    BN = 4096
    grid = (B, N // BM, N // BN)

    adj, rk2 = pl.pallas_call(
        _adj_kernel,
        grid=grid,
        in_specs=[
            pl.BlockSpec((1, BM, Z), lambda b, i, j: (b, i, 0)),
            pl.BlockSpec((1, BN, Z), lambda b, i, j: (b, j, 0)),
            pl.BlockSpec((1, Z), lambda b, i, j: (0, 0)),
        ],
        out_specs=[
            pl.BlockSpec((1, BM, BN), lambda b, i, j: (b, i, j)),
            pl.BlockSpec((1, Z), lambda b, i, j: (0, 0)),
        ],
        out_shape=[
            jax.ShapeDtypeStruct((B, N, N), jnp.float32),
            jax.ShapeDtypeStruct(rk_lgt.shape, jnp.float32),
        ],
        compiler_params=pltpu.CompilerParams(
            dimension_semantics=("parallel", "parallel", "parallel"),
        ),
    )(z1, z2, rk_lgt)

    return (adj, z1, z2, rk2)
